# baseline (speedup 1.0000x reference)
import functools

import jax
import jax.numpy as jnp
from jax import lax
from jax.experimental import pallas as pl
from jax.experimental.pallas import tpu as pltpu
from jax.experimental.pallas import tpu_sc as plsc

_B = 1024
_F = 4 * 64 * 64
_R = 64
_G = _B // _R
_NW = 32
_CHUNK = _B // _NW
_TBL = 1024
_F32MAX = 3.4028234663852886e38


def _sc_gather_body(stats_hbm, t_hbm, out_hbm, table_v, idx_v, out_v):
    wid = lax.axis_index("s") * 2 + lax.axis_index("c")
    base = wid * _CHUNK
    pltpu.sync_copy(stats_hbm, table_v)
    pltpu.sync_copy(t_hbm.at[pl.ds(base, _CHUNK)], idx_v)
    for j in range(_CHUNK // 16):
        iv = idx_v[pl.ds(j * 16, 16)]
        out_v[pl.ds(j * 16, 16)] = plsc.load_gather(table_v, [iv])
    pltpu.sync_copy(out_v, out_hbm.at[pl.ds(base, _CHUNK)])


@jax.jit
def _sc_gather(stats_pad, t32):
    mesh = plsc.VectorSubcoreMesh(core_axis_name="c", subcore_axis_name="s")
    k = functools.partial(
        pl.kernel,
        mesh=mesh,
        out_type=jax.ShapeDtypeStruct((_B,), jnp.float32),
        scratch_types=[
            pltpu.VMEM((_TBL,), jnp.float32),
            pltpu.VMEM((_CHUNK,), jnp.int32),
            pltpu.VMEM((_CHUNK,), jnp.float32),
        ],
        compiler_params=pltpu.CompilerParams(needs_layout_passes=False),
    )(_sc_gather_body)
    return k(stats_pad, t32)


def _rowsum(x):
    s = jnp.sum(x, axis=1)
    s = jnp.sum(s, axis=2)
    return jnp.sum(s, axis=1)


def _tc_body(stats_ref, g_ref, pred_ref, targ_ref,
             mse_ref, rmse_ref, tnorm_ref, pnorm_ref, r2_ref, loss_ref):
    i = pl.program_id(0)
    p = pred_ref[...]
    t = targ_ref[...]
    d = p - t
    inv = jnp.float32(1.0 / _F)
    mse = _rowsum(d * d) * inv
    pvar = _rowsum(p * p) * inv
    tvar = _rowsum(t * t) * inv
    mse_ref[0, 0, :] = mse
    rmse_ref[0, 0, :] = jnp.sqrt(mse)
    pnorm_ref[0, 0, :] = jnp.sqrt(pvar)
    tnorm_ref[0, 0, :] = jnp.sqrt(tvar)
    r2_ref[0, 0, :] = 1.0 - mse / tvar

    s = stats_ref[...]
    isn = s != s
    m = (jnp.sum(jnp.where(isn, 0.0, s))
         / jnp.sum(jnp.where(isn, jnp.float32(0.0), jnp.float32(1.0))))
    dist = m / g_ref[0, 0, :]
    dist = jnp.where(dist != dist, jnp.float32(1.0), dist)
    dist = jnp.where(dist == jnp.inf, jnp.float32(_F32MAX), dist)
    dist = jnp.where(dist == -jnp.inf, jnp.float32(-_F32MAX), dist)
    part = jnp.sum(mse * dist) * jnp.float32(1.0 / _B)

    @pl.when(i == 0)
    def _():
        loss_ref[...] = jnp.zeros((1, 1), jnp.float32)

    loss_ref[...] += jnp.reshape(part, (1, 1))


def _tc_stats(stats2d, g2d, pred2, targ2):
    row = jax.ShapeDtypeStruct((_G, 1, _R), jnp.float32)
    return pl.pallas_call(
        _tc_body,
        grid=(_G,),
        in_specs=[
            pl.BlockSpec((8, 128), lambda i: (0, 0)),
            pl.BlockSpec((1, 1, _R), lambda i: (i, 0, 0)),
            pl.BlockSpec((_R, 4, 64, 64), lambda i: (i, 0, 0, 0)),
            pl.BlockSpec((_R, 4, 64, 64), lambda i: (i, 0, 0, 0)),
        ],
        out_specs=[
            pl.BlockSpec((1, 1, _R), lambda i: (i, 0, 0)),
            pl.BlockSpec((1, 1, _R), lambda i: (i, 0, 0)),
            pl.BlockSpec((1, 1, _R), lambda i: (i, 0, 0)),
            pl.BlockSpec((1, 1, _R), lambda i: (i, 0, 0)),
            pl.BlockSpec((1, 1, _R), lambda i: (i, 0, 0)),
            pl.BlockSpec((1, 1), lambda i: (0, 0)),
        ],
        out_shape=[row, row, row, row, row,
                   jax.ShapeDtypeStruct((1, 1), jnp.float32)],
        compiler_params=pltpu.CompilerParams(
            dimension_semantics=("arbitrary",)),
    )(stats2d, g2d, pred2, targ2)


def kernel(pred, target, stats_mse, t):
    pred2 = pred
    targ2 = target
    stats_pad = jnp.concatenate(
        [stats_mse.astype(jnp.float32),
         jnp.full((_TBL - stats_mse.shape[0],), jnp.nan, jnp.float32)])
    t32 = jnp.asarray(t, jnp.int32)
    g = _sc_gather(stats_pad, t32)
    stats2d = jnp.reshape(stats_pad, (8, 128))
    g2d = jnp.reshape(g, (_G, 1, _R))
    mse, rmse, tnorm, pnorm, r2, loss = _tc_stats(stats2d, g2d, pred2, targ2)
    return (jnp.reshape(loss, ()),
            jnp.reshape(mse, (_B,)),
            jnp.reshape(rmse, (_B,)),
            jnp.reshape(tnorm, (_B,)),
            jnp.reshape(pnorm, (_B,)),
            jnp.reshape(r2, (_B,)))


# speedup vs baseline: 1.7613x; 1.7613x over previous
import functools

import jax
import jax.numpy as jnp
from jax import lax
from jax.experimental import pallas as pl
from jax.experimental.pallas import tpu as pltpu
from jax.experimental.pallas import tpu_sc as plsc

_B = 1024
_F = 4 * 64 * 64
_R = 64
_G = _B // _R
_NBUF = 4
_NW = 32
_CHUNK = _B // _NW
_TBL = 1024
_F32MAX = 3.4028234663852886e38


def _sc_gather_body(stats_hbm, t_hbm, out_hbm, table_v, idx_v, out_v):
    wid = lax.axis_index("s") * 2 + lax.axis_index("c")
    base = wid * _CHUNK
    pltpu.sync_copy(stats_hbm, table_v)
    pltpu.sync_copy(t_hbm.at[pl.ds(base, _CHUNK)], idx_v)
    for j in range(_CHUNK // 16):
        iv = idx_v[pl.ds(j * 16, 16)]
        out_v[pl.ds(j * 16, 16)] = plsc.load_gather(table_v, [iv])
    pltpu.sync_copy(out_v, out_hbm.at[pl.ds(base, _CHUNK)])


@jax.jit
def _sc_gather(stats_pad, t32):
    mesh = plsc.VectorSubcoreMesh(core_axis_name="c", subcore_axis_name="s")
    k = functools.partial(
        pl.kernel,
        mesh=mesh,
        out_type=jax.ShapeDtypeStruct((_B,), jnp.float32),
        scratch_types=[
            pltpu.VMEM((_TBL,), jnp.float32),
            pltpu.VMEM((_CHUNK,), jnp.int32),
            pltpu.VMEM((_CHUNK,), jnp.float32),
        ],
        compiler_params=pltpu.CompilerParams(needs_layout_passes=False),
    )(_sc_gather_body)
    return k(stats_pad, t32)


def _copy_chunk(hbm_ref, buf_ref, sem, chunk):
    return pltpu.make_async_copy(
        hbm_ref.at[pl.ds(chunk * _R, _R), :], buf_ref, sem)


def _tc_body(stats_ref, g_ref, pred_hbm, targ_hbm,
             mse_ref, rmse_ref, tnorm_ref, pnorm_ref, r2_ref, loss_ref,
             pbuf, tbuf, psem, tsem):
    i = pl.program_id(0)

    @pl.when(i == 0)
    def _prologue():
        for b in range(_NBUF):
            _copy_chunk(pred_hbm, pbuf.at[b], psem.at[b], b).start()
            _copy_chunk(targ_hbm, tbuf.at[b], tsem.at[b], b).start()

    slot = lax.rem(i, _NBUF)
    _copy_chunk(pred_hbm, pbuf.at[slot], psem.at[slot], i).wait()
    _copy_chunk(targ_hbm, tbuf.at[slot], tsem.at[slot], i).wait()

    p = pbuf[slot]
    t = tbuf[slot]
    d = p - t
    inv = jnp.float32(1.0 / _F)
    mse = jnp.sum(d * d, axis=1) * inv
    pvar = jnp.sum(p * p, axis=1) * inv
    tvar = jnp.sum(t * t, axis=1) * inv
    mse_ref[0, 0, :] = mse
    rmse_ref[0, 0, :] = jnp.sqrt(mse)
    pnorm_ref[0, 0, :] = jnp.sqrt(pvar)
    tnorm_ref[0, 0, :] = jnp.sqrt(tvar)
    r2_ref[0, 0, :] = 1.0 - mse / tvar

    s = stats_ref[...]
    isn = s != s
    m = (jnp.sum(jnp.where(isn, 0.0, s))
         / jnp.sum(jnp.where(isn, jnp.float32(0.0), jnp.float32(1.0))))
    dist = m / g_ref[0, 0, :]
    dist = jnp.where(dist != dist, jnp.float32(1.0), dist)
    dist = jnp.where(dist == jnp.inf, jnp.float32(_F32MAX), dist)
    dist = jnp.where(dist == -jnp.inf, jnp.float32(-_F32MAX), dist)
    part = jnp.sum(mse * dist) * jnp.float32(1.0 / _B)

    @pl.when(i == 0)
    def _init_loss():
        loss_ref[...] = jnp.zeros((1, 1), jnp.float32)

    loss_ref[...] += jnp.reshape(part, (1, 1))

    @pl.when(i + _NBUF < _G)
    def _refill():
        _copy_chunk(pred_hbm, pbuf.at[slot], psem.at[slot], i + _NBUF).start()
        _copy_chunk(targ_hbm, tbuf.at[slot], tsem.at[slot], i + _NBUF).start()


def _tc_stats(stats2d, g2d, pred2, targ2):
    row = jax.ShapeDtypeStruct((_G, 1, _R), jnp.float32)
    return pl.pallas_call(
        _tc_body,
        grid=(_G,),
        in_specs=[
            pl.BlockSpec((8, 128), lambda i: (0, 0)),
            pl.BlockSpec((1, 1, _R), lambda i: (i, 0, 0)),
            pl.BlockSpec(memory_space=pl.ANY),
            pl.BlockSpec(memory_space=pl.ANY),
        ],
        out_specs=[
            pl.BlockSpec((1, 1, _R), lambda i: (i, 0, 0)),
            pl.BlockSpec((1, 1, _R), lambda i: (i, 0, 0)),
            pl.BlockSpec((1, 1, _R), lambda i: (i, 0, 0)),
            pl.BlockSpec((1, 1, _R), lambda i: (i, 0, 0)),
            pl.BlockSpec((1, 1, _R), lambda i: (i, 0, 0)),
            pl.BlockSpec((1, 1), lambda i: (0, 0)),
        ],
        out_shape=[row, row, row, row, row,
                   jax.ShapeDtypeStruct((1, 1), jnp.float32)],
        scratch_shapes=[
            pltpu.VMEM((_NBUF, _R, _F), jnp.float32),
            pltpu.VMEM((_NBUF, _R, _F), jnp.float32),
            pltpu.SemaphoreType.DMA((_NBUF,)),
            pltpu.SemaphoreType.DMA((_NBUF,)),
        ],
        compiler_params=pltpu.CompilerParams(
            dimension_semantics=("arbitrary",)),
    )(stats2d, g2d, pred2, targ2)


def kernel(pred, target, stats_mse, t):
    pred2 = jnp.reshape(pred, (_B, _F))
    targ2 = jnp.reshape(target, (_B, _F))
    stats_pad = jnp.concatenate(
        [stats_mse.astype(jnp.float32),
         jnp.full((_TBL - stats_mse.shape[0],), jnp.nan, jnp.float32)])
    t32 = jnp.asarray(t, jnp.int32)
    g = _sc_gather(stats_pad, t32)
    stats2d = jnp.reshape(stats_pad, (8, 128))
    g2d = jnp.reshape(g, (_G, 1, _R))
    mse, rmse, tnorm, pnorm, r2, loss = _tc_stats(stats2d, g2d, pred2, targ2)
    return (jnp.reshape(loss, ()),
            jnp.reshape(mse, (_B,)),
            jnp.reshape(rmse, (_B,)),
            jnp.reshape(tnorm, (_B,)),
            jnp.reshape(pnorm, (_B,)),
            jnp.reshape(r2, (_B,)))


# speedup vs baseline: 4.9759x; 2.8251x over previous
import functools

import jax
import jax.numpy as jnp
from jax import lax
from jax.experimental import pallas as pl
from jax.experimental.pallas import tpu as pltpu
from jax.experimental.pallas import tpu_sc as plsc

_B = 1024
_F = 4 * 64 * 64
_C = 2048
_G = _F // _C
_NW = 32
_CHUNK = _B // _NW
_TBL = 1024
_F32MAX = 3.4028234663852886e38


def _sc_gather_body(stats_hbm, t_hbm, out_hbm, table_v, idx_v, out_v):
    wid = lax.axis_index("s") * 2 + lax.axis_index("c")
    base = wid * _CHUNK
    pltpu.sync_copy(stats_hbm, table_v)
    pltpu.sync_copy(t_hbm.at[pl.ds(base, _CHUNK)], idx_v)
    for j in range(_CHUNK // 16):
        iv = idx_v[pl.ds(j * 16, 16)]
        out_v[pl.ds(j * 16, 16)] = plsc.load_gather(table_v, [iv])
    pltpu.sync_copy(out_v, out_hbm.at[pl.ds(base, _CHUNK)])


@jax.jit
def _sc_gather(stats_pad, t32):
    mesh = plsc.VectorSubcoreMesh(core_axis_name="c", subcore_axis_name="s")
    k = functools.partial(
        pl.kernel,
        mesh=mesh,
        out_type=jax.ShapeDtypeStruct((_B,), jnp.float32),
        scratch_types=[
            pltpu.VMEM((_TBL,), jnp.float32),
            pltpu.VMEM((_CHUNK,), jnp.int32),
            pltpu.VMEM((_CHUNK,), jnp.float32),
        ],
        compiler_params=pltpu.CompilerParams(needs_layout_passes=False),
    )(_sc_gather_body)
    return k(stats_pad, t32)


def _tc_body(stats_ref, g_ref, pred_ref, targ_ref,
             mse_ref, rmse_ref, tnorm_ref, pnorm_ref, r2_ref, loss_ref,
             dacc, pacc, tacc):
    i = pl.program_id(0)
    p = pred_ref[...]
    t = targ_ref[...]
    d = p - t
    dsum = jnp.sum(d * d, axis=0)
    psum = jnp.sum(p * p, axis=0)
    tsum = jnp.sum(t * t, axis=0)

    @pl.when(i == 0)
    def _init():
        dacc[...] = dsum
        pacc[...] = psum
        tacc[...] = tsum

    @pl.when(i > 0)
    def _acc():
        dacc[...] += dsum
        pacc[...] += psum
        tacc[...] += tsum

    @pl.when(i == _G - 1)
    def _finalize():
        inv = jnp.float32(1.0 / _F)
        mse = dacc[...] * inv
        pvar = pacc[...] * inv
        tvar = tacc[...] * inv
        mse_ref[...] = mse
        rmse_ref[...] = jnp.sqrt(mse)
        pnorm_ref[...] = jnp.sqrt(pvar)
        tnorm_ref[...] = jnp.sqrt(tvar)
        r2_ref[...] = 1.0 - mse / tvar

        s = stats_ref[...]
        isn = s != s
        m = (jnp.sum(jnp.where(isn, 0.0, s))
             / jnp.sum(jnp.where(isn, jnp.float32(0.0), jnp.float32(1.0))))
        dist = m / g_ref[...]
        dist = jnp.where(dist != dist, jnp.float32(1.0), dist)
        dist = jnp.where(dist == jnp.inf, jnp.float32(_F32MAX), dist)
        dist = jnp.where(dist == -jnp.inf, jnp.float32(-_F32MAX), dist)
        loss_ref[...] = jnp.reshape(
            jnp.sum(mse * dist) * jnp.float32(1.0 / _B), (1, 1))


def _tc_stats(stats2d, g, pred_t, targ_t):
    vec = jax.ShapeDtypeStruct((_B,), jnp.float32)
    return pl.pallas_call(
        _tc_body,
        grid=(_G,),
        in_specs=[
            pl.BlockSpec((8, 128), lambda i: (0, 0)),
            pl.BlockSpec((_B,), lambda i: (0,)),
            pl.BlockSpec((_C, _B), lambda i: (i, 0)),
            pl.BlockSpec((_C, _B), lambda i: (i, 0)),
        ],
        out_specs=[
            pl.BlockSpec((_B,), lambda i: (0,)),
            pl.BlockSpec((_B,), lambda i: (0,)),
            pl.BlockSpec((_B,), lambda i: (0,)),
            pl.BlockSpec((_B,), lambda i: (0,)),
            pl.BlockSpec((_B,), lambda i: (0,)),
            pl.BlockSpec((1, 1), lambda i: (0, 0)),
        ],
        out_shape=[vec, vec, vec, vec, vec,
                   jax.ShapeDtypeStruct((1, 1), jnp.float32)],
        scratch_shapes=[
            pltpu.VMEM((_B,), jnp.float32),
            pltpu.VMEM((_B,), jnp.float32),
            pltpu.VMEM((_B,), jnp.float32),
        ],
        compiler_params=pltpu.CompilerParams(
            dimension_semantics=("arbitrary",)),
    )(stats2d, g, pred_t, targ_t)


def kernel(pred, target, stats_mse, t):
    pred_t = jnp.reshape(jnp.transpose(pred, (1, 2, 3, 0)), (_F, _B))
    targ_t = jnp.reshape(jnp.transpose(target, (1, 2, 3, 0)), (_F, _B))
    stats_pad = jnp.concatenate(
        [stats_mse.astype(jnp.float32),
         jnp.full((_TBL - stats_mse.shape[0],), jnp.nan, jnp.float32)])
    t32 = jnp.asarray(t, jnp.int32)
    g = _sc_gather(stats_pad, t32)
    stats2d = jnp.reshape(stats_pad, (8, 128))
    mse, rmse, tnorm, pnorm, r2, loss = _tc_stats(stats2d, g, pred_t, targ_t)
    return (jnp.reshape(loss, ()), mse, rmse, tnorm, pnorm, r2)


# speedup vs baseline: 5.4558x; 1.0964x over previous
import functools

import jax
import jax.numpy as jnp
from jax import lax
from jax.experimental import pallas as pl
from jax.experimental.pallas import tpu as pltpu
from jax.experimental.pallas import tpu_sc as plsc

_B = 1024
_F = 4 * 64 * 64
_C = 2048
_G = _F // _C
_NW = 32
_CHUNK = _B // _NW
_TBL = 1024
_F32MAX = 3.4028234663852886e38


def _sc_gather_body(stats_hbm, t_hbm, out_hbm, table_v, idx_v, out_v):
    wid = lax.axis_index("s") * 2 + lax.axis_index("c")
    base = wid * _CHUNK
    pltpu.sync_copy(stats_hbm, table_v)
    pltpu.sync_copy(t_hbm.at[pl.ds(base, _CHUNK)], idx_v)
    acc = jnp.zeros((16,), jnp.float32)
    cnt = jnp.zeros((16,), jnp.float32)
    for j in range(_TBL // 16):
        v = table_v[pl.ds(j * 16, 16)]
        isn = v != v
        acc = acc + jnp.where(isn, jnp.float32(0.0), v)
        cnt = cnt + jnp.where(isn, jnp.float32(0.0), jnp.float32(1.0))
    num = jnp.sum(acc)
    den = jnp.sum(cnt)
    mv = jnp.broadcast_to(num, (16,)) / jnp.broadcast_to(den, (16,))
    for j in range(_CHUNK // 16):
        iv = idx_v[pl.ds(j * 16, 16)]
        vals = plsc.load_gather(table_v, [iv])
        dist = mv / vals
        dist = jnp.where(dist != dist, jnp.float32(1.0), dist)
        dist = jnp.where(dist == jnp.inf, jnp.float32(_F32MAX), dist)
        dist = jnp.where(dist == -jnp.inf, jnp.float32(-_F32MAX), dist)
        out_v[pl.ds(j * 16, 16)] = dist
    pltpu.sync_copy(out_v, out_hbm.at[pl.ds(base, _CHUNK)])


@jax.jit
def _sc_gather(stats_pad, t32):
    mesh = plsc.VectorSubcoreMesh(core_axis_name="c", subcore_axis_name="s")
    k = functools.partial(
        pl.kernel,
        mesh=mesh,
        out_type=jax.ShapeDtypeStruct((_B,), jnp.float32),
        scratch_types=[
            pltpu.VMEM((_TBL,), jnp.float32),
            pltpu.VMEM((_CHUNK,), jnp.int32),
            pltpu.VMEM((_CHUNK,), jnp.float32),
        ],
        compiler_params=pltpu.CompilerParams(needs_layout_passes=False),
    )(_sc_gather_body)
    return k(stats_pad, t32)


def _tc_body(pred_ref, targ_ref,
             mse_ref, rmse_ref, tnorm_ref, pnorm_ref, r2_ref,
             dacc, pacc, tacc):
    i = pl.program_id(0)
    p = pred_ref[...]
    t = targ_ref[...]
    d = p - t
    dsum = jnp.sum(d * d, axis=0)
    psum = jnp.sum(p * p, axis=0)
    tsum = jnp.sum(t * t, axis=0)

    @pl.when(i == 0)
    def _init():
        dacc[...] = dsum
        pacc[...] = psum
        tacc[...] = tsum

    @pl.when(i > 0)
    def _acc():
        dacc[...] += dsum
        pacc[...] += psum
        tacc[...] += tsum

    @pl.when(i == _G - 1)
    def _finalize():
        inv = jnp.float32(1.0 / _F)
        mse = dacc[...] * inv
        pvar = pacc[...] * inv
        tvar = tacc[...] * inv
        mse_ref[...] = mse
        rmse_ref[...] = jnp.sqrt(mse)
        pnorm_ref[...] = jnp.sqrt(pvar)
        tnorm_ref[...] = jnp.sqrt(tvar)
        r2_ref[...] = 1.0 - mse / tvar


def _tc_stats(pred_t, targ_t):
    vec = jax.ShapeDtypeStruct((_B,), jnp.float32)
    return pl.pallas_call(
        _tc_body,
        grid=(_G,),
        in_specs=[
            pl.BlockSpec((_C, _B), lambda i: (i, 0)),
            pl.BlockSpec((_C, _B), lambda i: (i, 0)),
        ],
        out_specs=[
            pl.BlockSpec((_B,), lambda i: (0,)),
            pl.BlockSpec((_B,), lambda i: (0,)),
            pl.BlockSpec((_B,), lambda i: (0,)),
            pl.BlockSpec((_B,), lambda i: (0,)),
            pl.BlockSpec((_B,), lambda i: (0,)),
        ],
        out_shape=[vec, vec, vec, vec, vec],
        scratch_shapes=[
            pltpu.VMEM((_B,), jnp.float32),
            pltpu.VMEM((_B,), jnp.float32),
            pltpu.VMEM((_B,), jnp.float32),
        ],
        compiler_params=pltpu.CompilerParams(
            dimension_semantics=("arbitrary",)),
    )(pred_t, targ_t)


def _loss_body(mse_ref, dist_ref, loss_ref):
    loss_ref[...] = jnp.reshape(
        jnp.sum(mse_ref[...] * dist_ref[...]) * jnp.float32(1.0 / _B), (1, 1))


def _loss_combine(mse, dist):
    return pl.pallas_call(
        _loss_body,
        out_shape=jax.ShapeDtypeStruct((1, 1), jnp.float32),
    )(mse, dist)


def kernel(pred, target, stats_mse, t):
    pred_t = jnp.reshape(jnp.transpose(pred, (1, 2, 3, 0)), (_F, _B))
    targ_t = jnp.reshape(jnp.transpose(target, (1, 2, 3, 0)), (_F, _B))
    stats_pad = jnp.concatenate(
        [stats_mse.astype(jnp.float32),
         jnp.full((_TBL - stats_mse.shape[0],), jnp.nan, jnp.float32)])
    t32 = jnp.asarray(t, jnp.int32)
    dist = _sc_gather(stats_pad, t32)
    mse, rmse, tnorm, pnorm, r2 = _tc_stats(pred_t, targ_t)
    loss = _loss_combine(mse, dist)
    return (jnp.reshape(loss, ()), mse, rmse, tnorm, pnorm, r2)


# speedup vs baseline: 5.6783x; 1.0408x over previous
import functools

import jax
import jax.numpy as jnp
from jax import lax
from jax.experimental import pallas as pl
from jax.experimental.pallas import tpu as pltpu
from jax.experimental.pallas import tpu_sc as plsc

_B = 1024
_F = 4 * 64 * 64
_C = 2048
_G = _F // _C
_NW = 16
_CHUNK = _B // _NW
_TBL = 1000
_F32MAX = 3.4028234663852886e38


def _sc_gather_body(stats_hbm, t_hbm, out_hbm, table_v, idx_v, out_v):
    wid = lax.axis_index("s")
    base = wid * _CHUNK
    pltpu.sync_copy(stats_hbm, table_v.at[pl.ds(0, _TBL)])
    pltpu.sync_copy(t_hbm.at[pl.ds(base, _CHUNK)], idx_v)
    acc = jnp.zeros((16,), jnp.float32)
    cnt = jnp.zeros((16,), jnp.float32)
    lane = jnp.arange(16, dtype=jnp.int32)
    for j in range(_TBL // 16 + 1):
        v = table_v[pl.ds(j * 16, 16)]
        valid = (v == v) if (j + 1) * 16 <= _TBL else (
            (v == v) & (lane < _TBL - j * 16))
        acc = acc + jnp.where(valid, v, jnp.float32(0.0))
        cnt = cnt + jnp.where(valid, jnp.float32(1.0), jnp.float32(0.0))
    num = jnp.sum(acc)
    den = jnp.sum(cnt)
    mv = jnp.broadcast_to(num, (16,)) / jnp.broadcast_to(den, (16,))
    for j in range(_CHUNK // 16):
        iv = idx_v[pl.ds(j * 16, 16)]
        vals = plsc.load_gather(table_v, [iv])
        dist = mv / vals
        dist = jnp.where(dist != dist, jnp.float32(1.0), dist)
        dist = jnp.where(dist == jnp.inf, jnp.float32(_F32MAX), dist)
        dist = jnp.where(dist == -jnp.inf, jnp.float32(-_F32MAX), dist)
        out_v[pl.ds(j * 16, 16)] = dist
    pltpu.sync_copy(out_v, out_hbm.at[pl.ds(base, _CHUNK)])


@jax.jit
def _sc_gather(stats, t32):
    mesh = plsc.VectorSubcoreMesh(core_axis_name="c", subcore_axis_name="s",
                                  num_cores=1)
    k = functools.partial(
        pl.kernel,
        mesh=mesh,
        out_type=jax.ShapeDtypeStruct((_B,), jnp.float32),
        scratch_types=[
            pltpu.VMEM((_TBL + 16,), jnp.float32),
            pltpu.VMEM((_CHUNK,), jnp.int32),
            pltpu.VMEM((_CHUNK,), jnp.float32),
        ],
        compiler_params=pltpu.CompilerParams(needs_layout_passes=False),
    )(_sc_gather_body)
    return k(stats, t32)


def _tc_body(pred_ref, targ_ref,
             mse_ref, rmse_ref, tnorm_ref, pnorm_ref, r2_ref,
             dacc, pacc, tacc):
    i = pl.program_id(0)
    p = pred_ref[...]
    t = targ_ref[...]
    d = p - t
    dsum = jnp.sum(d * d, axis=0)
    psum = jnp.sum(p * p, axis=0)
    tsum = jnp.sum(t * t, axis=0)

    @pl.when(i == 0)
    def _init():
        dacc[...] = dsum
        pacc[...] = psum
        tacc[...] = tsum

    @pl.when(i > 0)
    def _acc():
        dacc[...] += dsum
        pacc[...] += psum
        tacc[...] += tsum

    @pl.when(i == _G - 1)
    def _finalize():
        inv = jnp.float32(1.0 / _F)
        mse = dacc[...] * inv
        pvar = pacc[...] * inv
        tvar = tacc[...] * inv
        mse_ref[...] = mse
        rmse_ref[...] = jnp.sqrt(mse)
        pnorm_ref[...] = jnp.sqrt(pvar)
        tnorm_ref[...] = jnp.sqrt(tvar)
        r2_ref[...] = 1.0 - mse / tvar


def _tc_stats(pred_t, targ_t):
    vec = jax.ShapeDtypeStruct((_B,), jnp.float32)
    return pl.pallas_call(
        _tc_body,
        grid=(_G,),
        in_specs=[
            pl.BlockSpec((_C, _B), lambda i: (i, 0)),
            pl.BlockSpec((_C, _B), lambda i: (i, 0)),
        ],
        out_specs=[
            pl.BlockSpec((_B,), lambda i: (0,)),
            pl.BlockSpec((_B,), lambda i: (0,)),
            pl.BlockSpec((_B,), lambda i: (0,)),
            pl.BlockSpec((_B,), lambda i: (0,)),
            pl.BlockSpec((_B,), lambda i: (0,)),
        ],
        out_shape=[vec, vec, vec, vec, vec],
        scratch_shapes=[
            pltpu.VMEM((_B,), jnp.float32),
            pltpu.VMEM((_B,), jnp.float32),
            pltpu.VMEM((_B,), jnp.float32),
        ],
        compiler_params=pltpu.CompilerParams(
            dimension_semantics=("arbitrary",)),
    )(pred_t, targ_t)


def _loss_body(mse_ref, dist_ref, loss_ref):
    loss_ref[...] = jnp.reshape(
        jnp.sum(mse_ref[...] * dist_ref[...]) * jnp.float32(1.0 / _B), (1, 1))


def _loss_combine(mse, dist):
    return pl.pallas_call(
        _loss_body,
        out_shape=jax.ShapeDtypeStruct((1, 1), jnp.float32),
    )(mse, dist)


def kernel(pred, target, stats_mse, t):
    pred_t = jnp.reshape(jnp.transpose(pred, (1, 2, 3, 0)), (_F, _B))
    targ_t = jnp.reshape(jnp.transpose(target, (1, 2, 3, 0)), (_F, _B))
    t32 = jnp.asarray(t, jnp.int32)
    dist = _sc_gather(stats_mse.astype(jnp.float32), t32)
    mse, rmse, tnorm, pnorm, r2 = _tc_stats(pred_t, targ_t)
    loss = _loss_combine(mse, dist)
    return (jnp.reshape(loss, ()), mse, rmse, tnorm, pnorm, r2)
